# baseline (speedup 1.0000x reference)
import functools

import jax
import jax.numpy as jnp
from jax import lax
from jax.experimental import pallas as pl
from jax.experimental.pallas import tpu as pltpu
from jax.experimental.pallas import tpu_sc as plsc

S, B, D, H, E, K, DFF = 2048, 1, 1024, 16, 8, 2, 2048
DH = D // H
T = S * B
RB = 256
NRB = T // RB
BLK = 64
P = T * K + E * BLK
NB = P // BLK
EPAD = 128


def _qkv_body(x_ref, y_ref, g_ref, b_ref, wq_ref, bq_ref, wk_ref, bk_ref,
              wv_ref, bv_ref, q_ref, k_ref, v_ref):
    g = g_ref[...]
    b = b_ref[...]

    def ln(u):
        mu = jnp.mean(u, axis=1, keepdims=True)
        d = u - mu
        var = jnp.mean(d * d, axis=1, keepdims=True)
        return d * lax.rsqrt(var + 1e-5) * g + b

    xn = ln(x_ref[...])
    yn = ln(y_ref[...])
    q_ref[...] = jnp.dot(xn, wq_ref[...], preferred_element_type=jnp.float32) + bq_ref[...]
    k_ref[...] = jnp.dot(yn, wk_ref[...], preferred_element_type=jnp.float32) + bk_ref[...]
    v_ref[...] = jnp.dot(yn, wv_ref[...], preferred_element_type=jnp.float32) + bv_ref[...]


def _qkv(xs, ys, g1, b1, Wq, bq, Wk, bk, Wv, bv):
    row = pl.BlockSpec((RB, D), lambda i: (i, 0))
    vec = pl.BlockSpec((1, D), lambda i: (0, 0))
    mat = pl.BlockSpec((D, D), lambda i: (0, 0))
    return pl.pallas_call(
        _qkv_body,
        grid=(NRB,),
        in_specs=[row, row, vec, vec, mat, vec, mat, vec, mat, vec],
        out_specs=[row, row, row],
        out_shape=[jax.ShapeDtypeStruct((T, D), jnp.float32)] * 3,
    )(xs, ys, g1, b1, Wq, bq, Wk, bk, Wv, bv)


def _attn_body(ro, ke, q_ref, k_ref, v_ref, o_ref):
    i = pl.program_id(1)
    q = q_ref[0] * jnp.float32(1.0 / (DH ** 0.5))
    s = lax.dot_general(q, k_ref[0], (((1,), (1,)), ((), ())),
                        preferred_element_type=jnp.float32)
    row = (ro + i) * RB + lax.broadcasted_iota(jnp.int32, (RB, ke), 0)
    col = lax.broadcasted_iota(jnp.int32, (RB, ke), 1)
    s = jnp.where(col <= row, s, jnp.float32(-1e9))
    m = jnp.max(s, axis=1, keepdims=True)
    p = jnp.exp(s - m)
    p = p / jnp.sum(p, axis=1, keepdims=True)
    o_ref[0] = jnp.dot(p, v_ref[0], preferred_element_type=jnp.float32)


_PR = 2


def _attention(q, k, v):
    parts = []
    for pi in range(NRB // _PR):
        ke = (pi + 1) * _PR * RB
        out = pl.pallas_call(
            functools.partial(_attn_body, pi * _PR, ke),
            grid=(H, _PR),
            in_specs=[
                pl.BlockSpec((1, RB, DH), lambda h, i, pi=pi: (h, pi * _PR + i, 0)),
                pl.BlockSpec((1, ke, DH), lambda h, i: (h, 0, 0)),
                pl.BlockSpec((1, ke, DH), lambda h, i: (h, 0, 0)),
            ],
            out_specs=pl.BlockSpec((1, RB, DH), lambda h, i: (h, i, 0)),
            out_shape=jax.ShapeDtypeStruct((H, _PR * RB, DH), jnp.float32),
        )(q, k, v)
        parts.append(out)
    return jnp.concatenate(parts, axis=1)


def _post_body(a_ref, x_ref, wo_ref, bo_ref, g2_ref, b2_ref, wg_ref,
               x1_ref, h2_ref, ti_ref, tg_ref):
    x1 = x_ref[...] + jnp.dot(a_ref[...], wo_ref[...],
                              preferred_element_type=jnp.float32) + bo_ref[...]
    x1_ref[...] = x1
    mu = jnp.mean(x1, axis=1, keepdims=True)
    d = x1 - mu
    var = jnp.mean(d * d, axis=1, keepdims=True)
    h2 = d * lax.rsqrt(var + 1e-5) * g2_ref[...] + b2_ref[...]
    h2_ref[...] = h2
    lg = jnp.dot(h2, wg_ref[...], preferred_element_type=jnp.float32)
    colid = lax.broadcasted_iota(jnp.int32, (RB, EPAD), 1)
    neg = jnp.float32(-1e30)
    lg = jnp.where(colid < E, lg, neg)
    m1 = jnp.max(lg, axis=1, keepdims=True)
    i1 = jnp.min(jnp.where(lg == m1, colid, EPAD), axis=1, keepdims=True)
    lg2 = jnp.where(colid == i1, neg, lg)
    m2 = jnp.max(lg2, axis=1, keepdims=True)
    i2 = jnp.min(jnp.where(lg2 == m2, colid, EPAD), axis=1, keepdims=True)
    e21 = jnp.exp(m2 - m1)
    den = 1.0 + e21
    ga = 1.0 / den
    gb = e21 / den
    ti_ref[...] = jnp.where(colid == 0, i1, jnp.where(colid == 1, i2, 0))
    tg_ref[...] = jnp.where(colid == 0, ga, jnp.where(colid == 1, gb, 0.0))


def _post(attn_out, xs, Wo, bo, g2, b2, Wg_pad):
    row = pl.BlockSpec((RB, D), lambda i: (i, 0))
    vec = pl.BlockSpec((1, D), lambda i: (0, 0))
    mat = pl.BlockSpec((D, D), lambda i: (0, 0))
    gspec = pl.BlockSpec((D, EPAD), lambda i: (0, 0))
    espec = pl.BlockSpec((RB, EPAD), lambda i: (i, 0))
    hspec = pl.BlockSpec((RB, D), lambda i: (i, 0))
    return pl.pallas_call(
        _post_body,
        grid=(NRB,),
        in_specs=[row, row, mat, vec, vec, vec, gspec],
        out_specs=[row, hspec, espec, espec],
        out_shape=[
            jax.ShapeDtypeStruct((T, D), jnp.float32),
            jax.ShapeDtypeStruct((T, D), jnp.float32),
            jax.ShapeDtypeStruct((T, EPAD), jnp.int32),
            jax.ShapeDtypeStruct((T, EPAD), jnp.float32),
        ],
    )(attn_out, xs, Wo, bo, g2, b2, Wg_pad)


_NC, _NS = 2, 16
_NW = _NC * _NS
_ROWS_W = P // _NW
_NCH = 3
_CH = _ROWS_W // _NCH


def _dispatch_body(h2_hbm, idx_hbm, out_hbm, idx_v, rows0_v, rows1_v,
                   gs0, gs1, os0, os1):
    wid = lax.axis_index("s") * _NC + lax.axis_index("c")
    base = wid * _ROWS_W
    pltpu.sync_copy(idx_hbm.at[pl.ds(base, _ROWS_W)], idx_v)
    bufs = (rows0_v, rows1_v)
    gsems = (gs0, gs1)
    osems = (os0, os1)
    gh = [None, None]
    oh = [None, None]
    for c in range(_NCH):
        b = c % 2
        if oh[b] is not None:
            oh[b].wait()
        gh[b] = pltpu.async_copy(
            h2_hbm.at[idx_v.at[pl.ds(c * _CH, _CH)]], bufs[b], gsems[b])
        if c >= 1:
            pb = (c - 1) % 2
            gh[pb].wait()
            oh[pb] = pltpu.async_copy(
                bufs[pb], out_hbm.at[pl.ds(base + (c - 1) * _CH, _CH)], osems[pb])
    lb = (_NCH - 1) % 2
    gh[lb].wait()
    oh[lb] = pltpu.async_copy(
        bufs[lb], out_hbm.at[pl.ds(base + (_NCH - 1) * _CH, _CH)], osems[lb])
    oh[0].wait()
    oh[1].wait()


def _dispatch(h2, src_tok):
    f = functools.partial(
        pl.kernel,
        mesh=plsc.VectorSubcoreMesh(core_axis_name="c", subcore_axis_name="s"),
        out_type=jax.ShapeDtypeStruct((P, D), jnp.float32),
        scratch_types=[
            pltpu.VMEM((_ROWS_W,), jnp.int32),
            pltpu.VMEM((_CH, D), jnp.float32),
            pltpu.VMEM((_CH, D), jnp.float32),
            pltpu.SemaphoreType.DMA,
            pltpu.SemaphoreType.DMA,
            pltpu.SemaphoreType.DMA,
            pltpu.SemaphoreType.DMA,
        ],
    )(_dispatch_body)
    return f(h2, src_tok)


def _moe_body(be_ref, hg_ref, w1_ref, b1_ref, w2_ref, b2_ref, eo_ref):
    eh = jnp.maximum(
        jnp.dot(hg_ref[...].astype(jnp.bfloat16), w1_ref[0],
                preferred_element_type=jnp.float32)
        + b1_ref[0], 0.0).astype(jnp.bfloat16)
    eo_ref[...] = jnp.dot(eh, w2_ref[0],
                          preferred_element_type=jnp.float32) + b2_ref[0]


def _moe(hg, be, W1, b1, W2, b2):
    grid_spec = pltpu.PrefetchScalarGridSpec(
        num_scalar_prefetch=1,
        grid=(NB,),
        in_specs=[
            pl.BlockSpec((BLK, D), lambda b, s: (b, 0)),
            pl.BlockSpec((1, D, DFF), lambda b, s: (s[b], 0, 0)),
            pl.BlockSpec((1, 1, DFF), lambda b, s: (s[b], 0, 0)),
            pl.BlockSpec((1, DFF, D), lambda b, s: (s[b], 0, 0)),
            pl.BlockSpec((1, 1, D), lambda b, s: (s[b], 0, 0)),
        ],
        out_specs=pl.BlockSpec((BLK, D), lambda b, s: (b, 0)),
    )
    return pl.pallas_call(
        _moe_body,
        grid_spec=grid_spec,
        out_shape=jax.ShapeDtypeStruct((P, D), jnp.float32),
    )(be, hg, W1, b1, W2, b2)


_GR_W = (2 * T) // _NW
_GNCH = 4
_GCH = _GR_W // _GNCH


def _cgather_body(eo_hbm, idx_hbm, out_hbm, idx_v, rows0_v, rows1_v,
                  gs0, gs1, os0, os1):
    wid = lax.axis_index("s") * _NC + lax.axis_index("c")
    base = wid * _GR_W
    pltpu.sync_copy(idx_hbm.at[pl.ds(base, _GR_W)], idx_v)
    bufs = (rows0_v, rows1_v)
    gsems = (gs0, gs1)
    osems = (os0, os1)
    gh = [None, None]
    oh = [None, None]
    for c in range(_GNCH):
        b = c % 2
        if oh[b] is not None:
            oh[b].wait()
        gh[b] = pltpu.async_copy(
            eo_hbm.at[idx_v.at[pl.ds(c * _GCH, _GCH)]], bufs[b], gsems[b])
        if c >= 1:
            pb = (c - 1) % 2
            gh[pb].wait()
            oh[pb] = pltpu.async_copy(
                bufs[pb], out_hbm.at[pl.ds(base + (c - 1) * _GCH, _GCH)], osems[pb])
    lb = (_GNCH - 1) % 2
    gh[lb].wait()
    oh[lb] = pltpu.async_copy(
        bufs[lb], out_hbm.at[pl.ds(base + (_GNCH - 1) * _GCH, _GCH)], osems[lb])
    oh[0].wait()
    oh[1].wait()


def _cgather(eo, dall):
    f = functools.partial(
        pl.kernel,
        mesh=plsc.VectorSubcoreMesh(core_axis_name="c", subcore_axis_name="s"),
        out_type=jax.ShapeDtypeStruct((2 * T, D), jnp.float32),
        scratch_types=[
            pltpu.VMEM((_GR_W,), jnp.int32),
            pltpu.VMEM((_GCH, D), jnp.float32),
            pltpu.VMEM((_GCH, D), jnp.float32),
            pltpu.SemaphoreType.DMA,
            pltpu.SemaphoreType.DMA,
            pltpu.SemaphoreType.DMA,
            pltpu.SemaphoreType.DMA,
        ],
    )(_cgather_body)
    return f(eo, dall)


def _final_body(x1_ref, r0_ref, r1_ref, tg_ref, o_ref):
    tg = tg_ref[...]
    g0 = tg[:, 0:1]
    g1 = tg[:, 1:2]
    o_ref[...] = x1_ref[...] + g0 * r0_ref[...] + g1 * r1_ref[...]


def _final(x1, R, tg):
    row = pl.BlockSpec((RB, D), lambda i: (i, 0))
    return pl.pallas_call(
        _final_body,
        grid=(NRB,),
        in_specs=[
            row,
            pl.BlockSpec((RB, D), lambda i: (i, 0)),
            pl.BlockSpec((RB, D), lambda i: (NRB + i, 0)),
            pl.BlockSpec((RB, EPAD), lambda i: (i, 0)),
        ],
        out_specs=row,
        out_shape=jax.ShapeDtypeStruct((T, D), jnp.float32),
    )(x1, R, R, tg)


def kernel(x, self_attn_input, halt_mask, layer_idx, Wq, bq, Wk, bk, Wv, bv,
           Wo, bo, ln1_g, ln1_b, ln2_g, ln2_b, Wg, W1, b1, W2, b2):
    del halt_mask, layer_idx
    xs = x.reshape(T, D)
    ys = self_attn_input.reshape(T, D)
    g1v = ln1_g.reshape(1, D)
    b1v = ln1_b.reshape(1, D)
    g2v = ln2_g.reshape(1, D)
    b2v = ln2_b.reshape(1, D)
    Wg_pad = jnp.zeros((D, EPAD), jnp.float32).at[:, :E].set(Wg)

    q, k, v = _qkv(xs, ys, g1v, b1v, Wq, bq.reshape(1, D), Wk, bk.reshape(1, D),
                   Wv, bv.reshape(1, D))
    q3 = q.reshape(T, H, DH).transpose(1, 0, 2)
    k3 = k.reshape(T, H, DH).transpose(1, 0, 2)
    v3 = v.reshape(T, H, DH).transpose(1, 0, 2)
    attn_out = _attention(q3, k3, v3).transpose(1, 0, 2).reshape(T, D)
    x1, h2, ti, tg = _post(attn_out, xs, Wo, bo.reshape(1, D), g2v, b2v, Wg_pad)

    ev = ti[:, :K].reshape(T * K)
    gk = tg[:, :K].reshape(T * K)
    oh = (ev[:, None] == jnp.arange(E)[None, :]).astype(jnp.int32)
    csum = jnp.cumsum(oh, axis=0)
    rank = jnp.take_along_axis(csum, ev[:, None], axis=1)[:, 0] - 1
    counts = csum[-1]
    padded = ((counts + BLK - 1) // BLK) * BLK
    offs = jnp.concatenate([jnp.zeros((1,), jnp.int32),
                            jnp.cumsum(padded)[:-1].astype(jnp.int32)])
    dest = offs[ev] + rank
    src_tok = jnp.zeros((P,), jnp.int32).at[dest].set(
        jnp.arange(T * K, dtype=jnp.int32) // K)
    bstart = jnp.arange(NB, dtype=jnp.int32) * BLK
    inside = (bstart[:, None] >= offs[None, :]) & \
             (bstart[:, None] < (offs + padded)[None, :])
    be = jnp.sum(jnp.where(inside, jnp.arange(E, dtype=jnp.int32)[None, :], 0),
                 axis=1).astype(jnp.int32)
    dall = dest.reshape(T, K).T.reshape(2 * T).astype(jnp.int32)

    hg = _dispatch(h2, src_tok)
    eo = _moe(hg, be, W1.astype(jnp.bfloat16), b1.reshape(E, 1, DFF),
              W2.astype(jnp.bfloat16), b2.reshape(E, 1, D))
    R = _cgather(eo, dall)
    out = _final(x1, R, tg)
    return out.reshape(S, B, D)


# speedup vs baseline: 1.0465x; 1.0465x over previous
import functools

import jax
import jax.numpy as jnp
from jax import lax
from jax.experimental import pallas as pl
from jax.experimental.pallas import tpu as pltpu
from jax.experimental.pallas import tpu_sc as plsc

S, B, D, H, E, K, DFF = 2048, 1, 1024, 16, 8, 2, 2048
DH = D // H
T = S * B
RB = 256
NRB = T // RB
BLK = 128
P = T * K + E * BLK
NB = P // BLK
EPAD = 128


def _qkv_body(x_ref, y_ref, g_ref, b_ref, wq_ref, bq_ref, wk_ref, bk_ref,
              wv_ref, bv_ref, q_ref, k_ref, v_ref):
    g = g_ref[...]
    b = b_ref[...]

    def ln(u):
        mu = jnp.mean(u, axis=1, keepdims=True)
        d = u - mu
        var = jnp.mean(d * d, axis=1, keepdims=True)
        return d * lax.rsqrt(var + 1e-5) * g + b

    xn = ln(x_ref[...])
    yn = ln(y_ref[...])
    q_ref[...] = jnp.dot(xn, wq_ref[...], preferred_element_type=jnp.float32) + bq_ref[...]
    k_ref[...] = jnp.dot(yn, wk_ref[...], preferred_element_type=jnp.float32) + bk_ref[...]
    v_ref[...] = jnp.dot(yn, wv_ref[...], preferred_element_type=jnp.float32) + bv_ref[...]


def _qkv(xs, ys, g1, b1, Wq, bq, Wk, bk, Wv, bv):
    row = pl.BlockSpec((RB, D), lambda i: (i, 0))
    vec = pl.BlockSpec((1, D), lambda i: (0, 0))
    mat = pl.BlockSpec((D, D), lambda i: (0, 0))
    return pl.pallas_call(
        _qkv_body,
        grid=(NRB,),
        in_specs=[row, row, vec, vec, mat, vec, mat, vec, mat, vec],
        out_specs=[row, row, row],
        out_shape=[jax.ShapeDtypeStruct((T, D), jnp.float32)] * 3,
    )(xs, ys, g1, b1, Wq, bq, Wk, bk, Wv, bv)


def _attn_body(ro, ke, q_ref, k_ref, v_ref, o_ref):
    i = pl.program_id(1)
    q = q_ref[0] * jnp.float32(1.0 / (DH ** 0.5))
    s = lax.dot_general(q, k_ref[0], (((1,), (1,)), ((), ())),
                        preferred_element_type=jnp.float32)
    row = (ro + i) * RB + lax.broadcasted_iota(jnp.int32, (RB, ke), 0)
    col = lax.broadcasted_iota(jnp.int32, (RB, ke), 1)
    s = jnp.where(col <= row, s, jnp.float32(-1e9))
    m = jnp.max(s, axis=1, keepdims=True)
    p = jnp.exp(s - m)
    p = p / jnp.sum(p, axis=1, keepdims=True)
    o_ref[0] = jnp.dot(p, v_ref[0], preferred_element_type=jnp.float32)


_PR = 2


def _attention(q, k, v):
    parts = []
    for pi in range(NRB // _PR):
        ke = (pi + 1) * _PR * RB
        out = pl.pallas_call(
            functools.partial(_attn_body, pi * _PR, ke),
            grid=(H, _PR),
            in_specs=[
                pl.BlockSpec((1, RB, DH), lambda h, i, pi=pi: (h, pi * _PR + i, 0)),
                pl.BlockSpec((1, ke, DH), lambda h, i: (h, 0, 0)),
                pl.BlockSpec((1, ke, DH), lambda h, i: (h, 0, 0)),
            ],
            out_specs=pl.BlockSpec((1, RB, DH), lambda h, i: (h, i, 0)),
            out_shape=jax.ShapeDtypeStruct((H, _PR * RB, DH), jnp.float32),
        )(q, k, v)
        parts.append(out)
    return jnp.concatenate(parts, axis=1)


def _post_body(a_ref, x_ref, wo_ref, bo_ref, g2_ref, b2_ref, wg_ref,
               x1_ref, h2_ref, ti_ref, tg_ref):
    x1 = x_ref[...] + jnp.dot(a_ref[...], wo_ref[...],
                              preferred_element_type=jnp.float32) + bo_ref[...]
    x1_ref[...] = x1
    mu = jnp.mean(x1, axis=1, keepdims=True)
    d = x1 - mu
    var = jnp.mean(d * d, axis=1, keepdims=True)
    h2 = d * lax.rsqrt(var + 1e-5) * g2_ref[...] + b2_ref[...]
    h2_ref[...] = h2
    lg = jnp.dot(h2, wg_ref[...], preferred_element_type=jnp.float32)
    colid = lax.broadcasted_iota(jnp.int32, (RB, EPAD), 1)
    neg = jnp.float32(-1e30)
    lg = jnp.where(colid < E, lg, neg)
    m1 = jnp.max(lg, axis=1, keepdims=True)
    i1 = jnp.min(jnp.where(lg == m1, colid, EPAD), axis=1, keepdims=True)
    lg2 = jnp.where(colid == i1, neg, lg)
    m2 = jnp.max(lg2, axis=1, keepdims=True)
    i2 = jnp.min(jnp.where(lg2 == m2, colid, EPAD), axis=1, keepdims=True)
    e21 = jnp.exp(m2 - m1)
    den = 1.0 + e21
    ga = 1.0 / den
    gb = e21 / den
    ti_ref[...] = jnp.where(colid == 0, i1, jnp.where(colid == 1, i2, 0))
    tg_ref[...] = jnp.where(colid == 0, ga, jnp.where(colid == 1, gb, 0.0))


def _post(attn_out, xs, Wo, bo, g2, b2, Wg_pad):
    row = pl.BlockSpec((RB, D), lambda i: (i, 0))
    vec = pl.BlockSpec((1, D), lambda i: (0, 0))
    mat = pl.BlockSpec((D, D), lambda i: (0, 0))
    gspec = pl.BlockSpec((D, EPAD), lambda i: (0, 0))
    espec = pl.BlockSpec((RB, EPAD), lambda i: (i, 0))
    hspec = pl.BlockSpec((RB, D), lambda i: (i, 0))
    return pl.pallas_call(
        _post_body,
        grid=(NRB,),
        in_specs=[row, row, mat, vec, vec, vec, gspec],
        out_specs=[row, hspec, espec, espec],
        out_shape=[
            jax.ShapeDtypeStruct((T, D), jnp.float32),
            jax.ShapeDtypeStruct((T, D), jnp.float32),
            jax.ShapeDtypeStruct((T, EPAD), jnp.int32),
            jax.ShapeDtypeStruct((T, EPAD), jnp.float32),
        ],
    )(attn_out, xs, Wo, bo, g2, b2, Wg_pad)


_NC, _NS = 2, 16
_NW = _NC * _NS
_ROWS_W = P // _NW
_NCH = 4
_CH = _ROWS_W // _NCH


def _dispatch_body(h2_hbm, idx_hbm, out_hbm, idx_v, rows0_v, rows1_v,
                   gs0, gs1, os0, os1):
    wid = lax.axis_index("s") * _NC + lax.axis_index("c")
    base = wid * _ROWS_W
    pltpu.sync_copy(idx_hbm.at[pl.ds(base, _ROWS_W)], idx_v)
    bufs = (rows0_v, rows1_v)
    gsems = (gs0, gs1)
    osems = (os0, os1)
    gh = [None, None]
    oh = [None, None]
    for c in range(_NCH):
        b = c % 2
        if oh[b] is not None:
            oh[b].wait()
        gh[b] = pltpu.async_copy(
            h2_hbm.at[idx_v.at[pl.ds(c * _CH, _CH)]], bufs[b], gsems[b])
        if c >= 1:
            pb = (c - 1) % 2
            gh[pb].wait()
            oh[pb] = pltpu.async_copy(
                bufs[pb], out_hbm.at[pl.ds(base + (c - 1) * _CH, _CH)], osems[pb])
    lb = (_NCH - 1) % 2
    gh[lb].wait()
    oh[lb] = pltpu.async_copy(
        bufs[lb], out_hbm.at[pl.ds(base + (_NCH - 1) * _CH, _CH)], osems[lb])
    oh[0].wait()
    oh[1].wait()


def _dispatch(h2, src_tok):
    f = functools.partial(
        pl.kernel,
        mesh=plsc.VectorSubcoreMesh(core_axis_name="c", subcore_axis_name="s"),
        out_type=jax.ShapeDtypeStruct((P, D), jnp.float32),
        scratch_types=[
            pltpu.VMEM((_ROWS_W,), jnp.int32),
            pltpu.VMEM((_CH, D), jnp.float32),
            pltpu.VMEM((_CH, D), jnp.float32),
            pltpu.SemaphoreType.DMA,
            pltpu.SemaphoreType.DMA,
            pltpu.SemaphoreType.DMA,
            pltpu.SemaphoreType.DMA,
        ],
    )(_dispatch_body)
    return f(h2, src_tok)


def _moe_body(be_ref, hg_ref, w1_ref, b1_ref, w2_ref, b2_ref, eo_ref):
    eh = jnp.maximum(
        jnp.dot(hg_ref[...].astype(jnp.bfloat16), w1_ref[0],
                preferred_element_type=jnp.float32)
        + b1_ref[0], 0.0).astype(jnp.bfloat16)
    eo_ref[...] = jnp.dot(eh, w2_ref[0],
                          preferred_element_type=jnp.float32) + b2_ref[0]


def _moe(hg, be, W1, b1, W2, b2):
    grid_spec = pltpu.PrefetchScalarGridSpec(
        num_scalar_prefetch=1,
        grid=(NB,),
        in_specs=[
            pl.BlockSpec((BLK, D), lambda b, s: (b, 0)),
            pl.BlockSpec((1, D, DFF), lambda b, s: (s[b], 0, 0)),
            pl.BlockSpec((1, 1, DFF), lambda b, s: (s[b], 0, 0)),
            pl.BlockSpec((1, DFF, D), lambda b, s: (s[b], 0, 0)),
            pl.BlockSpec((1, 1, D), lambda b, s: (s[b], 0, 0)),
        ],
        out_specs=pl.BlockSpec((BLK, D), lambda b, s: (b, 0)),
    )
    return pl.pallas_call(
        _moe_body,
        grid_spec=grid_spec,
        out_shape=jax.ShapeDtypeStruct((P, D), jnp.float32),
    )(be, hg, W1, b1, W2, b2)


_GR_W = (2 * T) // _NW
_GNCH = 4
_GCH = _GR_W // _GNCH


def _cgather_body(eo_hbm, idx_hbm, out_hbm, idx_v, rows0_v, rows1_v,
                  gs0, gs1, os0, os1):
    wid = lax.axis_index("s") * _NC + lax.axis_index("c")
    base = wid * _GR_W
    pltpu.sync_copy(idx_hbm.at[pl.ds(base, _GR_W)], idx_v)
    bufs = (rows0_v, rows1_v)
    gsems = (gs0, gs1)
    osems = (os0, os1)
    gh = [None, None]
    oh = [None, None]
    for c in range(_GNCH):
        b = c % 2
        if oh[b] is not None:
            oh[b].wait()
        gh[b] = pltpu.async_copy(
            eo_hbm.at[idx_v.at[pl.ds(c * _GCH, _GCH)]], bufs[b], gsems[b])
        if c >= 1:
            pb = (c - 1) % 2
            gh[pb].wait()
            oh[pb] = pltpu.async_copy(
                bufs[pb], out_hbm.at[pl.ds(base + (c - 1) * _GCH, _GCH)], osems[pb])
    lb = (_GNCH - 1) % 2
    gh[lb].wait()
    oh[lb] = pltpu.async_copy(
        bufs[lb], out_hbm.at[pl.ds(base + (_GNCH - 1) * _GCH, _GCH)], osems[lb])
    oh[0].wait()
    oh[1].wait()


def _cgather(eo, dall):
    f = functools.partial(
        pl.kernel,
        mesh=plsc.VectorSubcoreMesh(core_axis_name="c", subcore_axis_name="s"),
        out_type=jax.ShapeDtypeStruct((2 * T, D), jnp.float32),
        scratch_types=[
            pltpu.VMEM((_GR_W,), jnp.int32),
            pltpu.VMEM((_GCH, D), jnp.float32),
            pltpu.VMEM((_GCH, D), jnp.float32),
            pltpu.SemaphoreType.DMA,
            pltpu.SemaphoreType.DMA,
            pltpu.SemaphoreType.DMA,
            pltpu.SemaphoreType.DMA,
        ],
    )(_cgather_body)
    return f(eo, dall)


def _final_body(x1_ref, r0_ref, r1_ref, tg_ref, o_ref):
    tg = tg_ref[...]
    g0 = tg[:, 0:1]
    g1 = tg[:, 1:2]
    o_ref[...] = x1_ref[...] + g0 * r0_ref[...] + g1 * r1_ref[...]


def _final(x1, R, tg):
    row = pl.BlockSpec((RB, D), lambda i: (i, 0))
    return pl.pallas_call(
        _final_body,
        grid=(NRB,),
        in_specs=[
            row,
            pl.BlockSpec((RB, D), lambda i: (i, 0)),
            pl.BlockSpec((RB, D), lambda i: (NRB + i, 0)),
            pl.BlockSpec((RB, EPAD), lambda i: (i, 0)),
        ],
        out_specs=row,
        out_shape=jax.ShapeDtypeStruct((T, D), jnp.float32),
    )(x1, R, R, tg)


def kernel(x, self_attn_input, halt_mask, layer_idx, Wq, bq, Wk, bk, Wv, bv,
           Wo, bo, ln1_g, ln1_b, ln2_g, ln2_b, Wg, W1, b1, W2, b2):
    del halt_mask, layer_idx
    xs = x.reshape(T, D)
    ys = self_attn_input.reshape(T, D)
    g1v = ln1_g.reshape(1, D)
    b1v = ln1_b.reshape(1, D)
    g2v = ln2_g.reshape(1, D)
    b2v = ln2_b.reshape(1, D)
    Wg_pad = jnp.zeros((D, EPAD), jnp.float32).at[:, :E].set(Wg)

    q, k, v = _qkv(xs, ys, g1v, b1v, Wq, bq.reshape(1, D), Wk, bk.reshape(1, D),
                   Wv, bv.reshape(1, D))
    q3 = q.reshape(T, H, DH).transpose(1, 0, 2)
    k3 = k.reshape(T, H, DH).transpose(1, 0, 2)
    v3 = v.reshape(T, H, DH).transpose(1, 0, 2)
    attn_out = _attention(q3, k3, v3).transpose(1, 0, 2).reshape(T, D)
    x1, h2, ti, tg = _post(attn_out, xs, Wo, bo.reshape(1, D), g2v, b2v, Wg_pad)

    ev = ti[:, :K].reshape(T * K)
    gk = tg[:, :K].reshape(T * K)
    oh = (ev[:, None] == jnp.arange(E)[None, :]).astype(jnp.int32)
    csum = jnp.cumsum(oh, axis=0)
    rank = jnp.take_along_axis(csum, ev[:, None], axis=1)[:, 0] - 1
    counts = csum[-1]
    padded = ((counts + BLK - 1) // BLK) * BLK
    offs = jnp.concatenate([jnp.zeros((1,), jnp.int32),
                            jnp.cumsum(padded)[:-1].astype(jnp.int32)])
    dest = offs[ev] + rank
    src_tok = jnp.zeros((P,), jnp.int32).at[dest].set(
        jnp.arange(T * K, dtype=jnp.int32) // K)
    bstart = jnp.arange(NB, dtype=jnp.int32) * BLK
    inside = (bstart[:, None] >= offs[None, :]) & \
             (bstart[:, None] < (offs + padded)[None, :])
    be = jnp.sum(jnp.where(inside, jnp.arange(E, dtype=jnp.int32)[None, :], 0),
                 axis=1).astype(jnp.int32)
    dall = dest.reshape(T, K).T.reshape(2 * T).astype(jnp.int32)

    hg = _dispatch(h2, src_tok)
    eo = _moe(hg, be, W1.astype(jnp.bfloat16), b1.reshape(E, 1, DFF),
              W2.astype(jnp.bfloat16), b2.reshape(E, 1, D))
    R = _cgather(eo, dall)
    out = _final(x1, R, tg)
    return out.reshape(S, B, D)


# speedup vs baseline: 1.0494x; 1.0027x over previous
import functools

import jax
import jax.numpy as jnp
from jax import lax
from jax.experimental import pallas as pl
from jax.experimental.pallas import tpu as pltpu
from jax.experimental.pallas import tpu_sc as plsc

S, B, D, H, E, K, DFF = 2048, 1, 1024, 16, 8, 2, 2048
DH = D // H
T = S * B
RB = 256
NRB = T // RB
BLK = 128
P = T * K + E * BLK
NB = P // BLK
EPAD = 128


def _qkv_body(x_ref, y_ref, g_ref, b_ref, wq_ref, bq_ref, wk_ref, bk_ref,
              wv_ref, bv_ref, q_ref, k_ref, v_ref):
    g = g_ref[...]
    b = b_ref[...]

    def ln(u):
        mu = jnp.mean(u, axis=1, keepdims=True)
        d = u - mu
        var = jnp.mean(d * d, axis=1, keepdims=True)
        return d * lax.rsqrt(var + 1e-5) * g + b

    xn = ln(x_ref[...])
    yn = ln(y_ref[...])
    q_ref[...] = jnp.dot(xn, wq_ref[...], preferred_element_type=jnp.float32) + bq_ref[...]
    k_ref[...] = jnp.dot(yn, wk_ref[...], preferred_element_type=jnp.float32) + bk_ref[...]
    v_ref[...] = jnp.dot(yn, wv_ref[...], preferred_element_type=jnp.float32) + bv_ref[...]


def _qkv(xs, ys, g1, b1, Wq, bq, Wk, bk, Wv, bv):
    row = pl.BlockSpec((RB, D), lambda i: (i, 0))
    vec = pl.BlockSpec((1, D), lambda i: (0, 0))
    mat = pl.BlockSpec((D, D), lambda i: (0, 0))
    return pl.pallas_call(
        _qkv_body,
        grid=(NRB,),
        in_specs=[row, row, vec, vec, mat, vec, mat, vec, mat, vec],
        out_specs=[row, row, row],
        out_shape=[jax.ShapeDtypeStruct((T, D), jnp.float32)] * 3,
    )(xs, ys, g1, b1, Wq, bq, Wk, bk, Wv, bv)


def _attn_body(ro, ke, q_ref, k_ref, v_ref, o_ref):
    i = pl.program_id(1)
    q = q_ref[0] * jnp.float32(1.0 / (DH ** 0.5))
    s = lax.dot_general(q, k_ref[0], (((1,), (1,)), ((), ())),
                        preferred_element_type=jnp.float32)
    row = (ro + i) * RB + lax.broadcasted_iota(jnp.int32, (RB, ke), 0)
    col = lax.broadcasted_iota(jnp.int32, (RB, ke), 1)
    s = jnp.where(col <= row, s, jnp.float32(-1e9))
    m = jnp.max(s, axis=1, keepdims=True)
    p = jnp.exp(s - m)
    p = p / jnp.sum(p, axis=1, keepdims=True)
    o_ref[0] = jnp.dot(p, v_ref[0], preferred_element_type=jnp.float32)


_PR = 1


def _attention(q, k, v):
    parts = []
    for pi in range(NRB // _PR):
        ke = (pi + 1) * _PR * RB
        out = pl.pallas_call(
            functools.partial(_attn_body, pi * _PR, ke),
            grid=(H, _PR),
            in_specs=[
                pl.BlockSpec((1, RB, DH), lambda h, i, pi=pi: (h, pi * _PR + i, 0)),
                pl.BlockSpec((1, ke, DH), lambda h, i: (h, 0, 0)),
                pl.BlockSpec((1, ke, DH), lambda h, i: (h, 0, 0)),
            ],
            out_specs=pl.BlockSpec((1, RB, DH), lambda h, i: (h, i, 0)),
            out_shape=jax.ShapeDtypeStruct((H, _PR * RB, DH), jnp.float32),
        )(q, k, v)
        parts.append(out)
    return jnp.concatenate(parts, axis=1)


def _post_body(a_ref, x_ref, wo_ref, bo_ref, g2_ref, b2_ref, wg_ref,
               x1_ref, h2_ref, ti_ref, tg_ref):
    x1 = x_ref[...] + jnp.dot(a_ref[...], wo_ref[...],
                              preferred_element_type=jnp.float32) + bo_ref[...]
    x1_ref[...] = x1
    mu = jnp.mean(x1, axis=1, keepdims=True)
    d = x1 - mu
    var = jnp.mean(d * d, axis=1, keepdims=True)
    h2 = d * lax.rsqrt(var + 1e-5) * g2_ref[...] + b2_ref[...]
    h2_ref[...] = h2
    lg = jnp.dot(h2, wg_ref[...], preferred_element_type=jnp.float32)
    colid = lax.broadcasted_iota(jnp.int32, (RB, EPAD), 1)
    neg = jnp.float32(-1e30)
    lg = jnp.where(colid < E, lg, neg)
    m1 = jnp.max(lg, axis=1, keepdims=True)
    i1 = jnp.min(jnp.where(lg == m1, colid, EPAD), axis=1, keepdims=True)
    lg2 = jnp.where(colid == i1, neg, lg)
    m2 = jnp.max(lg2, axis=1, keepdims=True)
    i2 = jnp.min(jnp.where(lg2 == m2, colid, EPAD), axis=1, keepdims=True)
    e21 = jnp.exp(m2 - m1)
    den = 1.0 + e21
    ga = 1.0 / den
    gb = e21 / den
    ti_ref[...] = jnp.where(colid == 0, i1, jnp.where(colid == 1, i2, 0))
    tg_ref[...] = jnp.where(colid == 0, ga, jnp.where(colid == 1, gb, 0.0))


def _post(attn_out, xs, Wo, bo, g2, b2, Wg_pad):
    row = pl.BlockSpec((RB, D), lambda i: (i, 0))
    vec = pl.BlockSpec((1, D), lambda i: (0, 0))
    mat = pl.BlockSpec((D, D), lambda i: (0, 0))
    gspec = pl.BlockSpec((D, EPAD), lambda i: (0, 0))
    espec = pl.BlockSpec((RB, EPAD), lambda i: (i, 0))
    hspec = pl.BlockSpec((RB, D), lambda i: (i, 0))
    return pl.pallas_call(
        _post_body,
        grid=(NRB,),
        in_specs=[row, row, mat, vec, vec, vec, gspec],
        out_specs=[row, hspec, espec, espec],
        out_shape=[
            jax.ShapeDtypeStruct((T, D), jnp.float32),
            jax.ShapeDtypeStruct((T, D), jnp.float32),
            jax.ShapeDtypeStruct((T, EPAD), jnp.int32),
            jax.ShapeDtypeStruct((T, EPAD), jnp.float32),
        ],
    )(attn_out, xs, Wo, bo, g2, b2, Wg_pad)


_NC, _NS = 2, 16
_NW = _NC * _NS
_ROWS_W = P // _NW
_NCH = 4
_CH = _ROWS_W // _NCH


def _dispatch_body(h2_hbm, idx_hbm, out_hbm, idx_v, rows0_v, rows1_v,
                   gs0, gs1, os0, os1):
    wid = lax.axis_index("s") * _NC + lax.axis_index("c")
    base = wid * _ROWS_W
    pltpu.sync_copy(idx_hbm.at[pl.ds(base, _ROWS_W)], idx_v)
    bufs = (rows0_v, rows1_v)
    gsems = (gs0, gs1)
    osems = (os0, os1)
    gh = [None, None]
    oh = [None, None]
    for c in range(_NCH):
        b = c % 2
        if oh[b] is not None:
            oh[b].wait()
        gh[b] = pltpu.async_copy(
            h2_hbm.at[idx_v.at[pl.ds(c * _CH, _CH)]], bufs[b], gsems[b])
        if c >= 1:
            pb = (c - 1) % 2
            gh[pb].wait()
            oh[pb] = pltpu.async_copy(
                bufs[pb], out_hbm.at[pl.ds(base + (c - 1) * _CH, _CH)], osems[pb])
    lb = (_NCH - 1) % 2
    gh[lb].wait()
    oh[lb] = pltpu.async_copy(
        bufs[lb], out_hbm.at[pl.ds(base + (_NCH - 1) * _CH, _CH)], osems[lb])
    oh[0].wait()
    oh[1].wait()


def _dispatch(h2, src_tok):
    f = functools.partial(
        pl.kernel,
        mesh=plsc.VectorSubcoreMesh(core_axis_name="c", subcore_axis_name="s"),
        out_type=jax.ShapeDtypeStruct((P, D), jnp.float32),
        scratch_types=[
            pltpu.VMEM((_ROWS_W,), jnp.int32),
            pltpu.VMEM((_CH, D), jnp.float32),
            pltpu.VMEM((_CH, D), jnp.float32),
            pltpu.SemaphoreType.DMA,
            pltpu.SemaphoreType.DMA,
            pltpu.SemaphoreType.DMA,
            pltpu.SemaphoreType.DMA,
        ],
    )(_dispatch_body)
    return f(h2, src_tok)


def _moe_body(be_ref, hg_ref, w1_ref, b1_ref, w2_ref, b2_ref, eo_ref):
    eh = jnp.maximum(
        jnp.dot(hg_ref[...].astype(jnp.bfloat16), w1_ref[0],
                preferred_element_type=jnp.float32)
        + b1_ref[0], 0.0).astype(jnp.bfloat16)
    eo_ref[...] = jnp.dot(eh, w2_ref[0],
                          preferred_element_type=jnp.float32) + b2_ref[0]


def _moe(hg, be, W1, b1, W2, b2):
    grid_spec = pltpu.PrefetchScalarGridSpec(
        num_scalar_prefetch=1,
        grid=(NB,),
        in_specs=[
            pl.BlockSpec((BLK, D), lambda b, s: (b, 0)),
            pl.BlockSpec((1, D, DFF), lambda b, s: (s[b], 0, 0)),
            pl.BlockSpec((1, 1, DFF), lambda b, s: (s[b], 0, 0)),
            pl.BlockSpec((1, DFF, D), lambda b, s: (s[b], 0, 0)),
            pl.BlockSpec((1, 1, D), lambda b, s: (s[b], 0, 0)),
        ],
        out_specs=pl.BlockSpec((BLK, D), lambda b, s: (b, 0)),
    )
    return pl.pallas_call(
        _moe_body,
        grid_spec=grid_spec,
        out_shape=jax.ShapeDtypeStruct((P, D), jnp.float32),
    )(be, hg, W1, b1, W2, b2)


_GR_W = (2 * T) // _NW
_GNCH = 4
_GCH = _GR_W // _GNCH


def _cgather_body(eo_hbm, idx_hbm, out_hbm, idx_v, rows0_v, rows1_v,
                  gs0, gs1, os0, os1):
    wid = lax.axis_index("s") * _NC + lax.axis_index("c")
    base = wid * _GR_W
    pltpu.sync_copy(idx_hbm.at[pl.ds(base, _GR_W)], idx_v)
    bufs = (rows0_v, rows1_v)
    gsems = (gs0, gs1)
    osems = (os0, os1)
    gh = [None, None]
    oh = [None, None]
    for c in range(_GNCH):
        b = c % 2
        if oh[b] is not None:
            oh[b].wait()
        gh[b] = pltpu.async_copy(
            eo_hbm.at[idx_v.at[pl.ds(c * _GCH, _GCH)]], bufs[b], gsems[b])
        if c >= 1:
            pb = (c - 1) % 2
            gh[pb].wait()
            oh[pb] = pltpu.async_copy(
                bufs[pb], out_hbm.at[pl.ds(base + (c - 1) * _GCH, _GCH)], osems[pb])
    lb = (_GNCH - 1) % 2
    gh[lb].wait()
    oh[lb] = pltpu.async_copy(
        bufs[lb], out_hbm.at[pl.ds(base + (_GNCH - 1) * _GCH, _GCH)], osems[lb])
    oh[0].wait()
    oh[1].wait()


def _cgather(eo, dall):
    f = functools.partial(
        pl.kernel,
        mesh=plsc.VectorSubcoreMesh(core_axis_name="c", subcore_axis_name="s"),
        out_type=jax.ShapeDtypeStruct((2 * T, D), jnp.float32),
        scratch_types=[
            pltpu.VMEM((_GR_W,), jnp.int32),
            pltpu.VMEM((_GCH, D), jnp.float32),
            pltpu.VMEM((_GCH, D), jnp.float32),
            pltpu.SemaphoreType.DMA,
            pltpu.SemaphoreType.DMA,
            pltpu.SemaphoreType.DMA,
            pltpu.SemaphoreType.DMA,
        ],
    )(_cgather_body)
    return f(eo, dall)


def _final_body(x1_ref, r0_ref, r1_ref, tg_ref, o_ref):
    tg = tg_ref[...]
    g0 = tg[:, 0:1]
    g1 = tg[:, 1:2]
    o_ref[...] = x1_ref[...] + g0 * r0_ref[...] + g1 * r1_ref[...]


def _final(x1, R, tg):
    row = pl.BlockSpec((RB, D), lambda i: (i, 0))
    return pl.pallas_call(
        _final_body,
        grid=(NRB,),
        in_specs=[
            row,
            pl.BlockSpec((RB, D), lambda i: (i, 0)),
            pl.BlockSpec((RB, D), lambda i: (NRB + i, 0)),
            pl.BlockSpec((RB, EPAD), lambda i: (i, 0)),
        ],
        out_specs=row,
        out_shape=jax.ShapeDtypeStruct((T, D), jnp.float32),
    )(x1, R, R, tg)


def kernel(x, self_attn_input, halt_mask, layer_idx, Wq, bq, Wk, bk, Wv, bv,
           Wo, bo, ln1_g, ln1_b, ln2_g, ln2_b, Wg, W1, b1, W2, b2):
    del halt_mask, layer_idx
    xs = x.reshape(T, D)
    ys = self_attn_input.reshape(T, D)
    g1v = ln1_g.reshape(1, D)
    b1v = ln1_b.reshape(1, D)
    g2v = ln2_g.reshape(1, D)
    b2v = ln2_b.reshape(1, D)
    Wg_pad = jnp.zeros((D, EPAD), jnp.float32).at[:, :E].set(Wg)

    q, k, v = _qkv(xs, ys, g1v, b1v, Wq, bq.reshape(1, D), Wk, bk.reshape(1, D),
                   Wv, bv.reshape(1, D))
    q3 = q.reshape(T, H, DH).transpose(1, 0, 2)
    k3 = k.reshape(T, H, DH).transpose(1, 0, 2)
    v3 = v.reshape(T, H, DH).transpose(1, 0, 2)
    attn_out = _attention(q3, k3, v3).transpose(1, 0, 2).reshape(T, D)
    x1, h2, ti, tg = _post(attn_out, xs, Wo, bo.reshape(1, D), g2v, b2v, Wg_pad)

    ev = ti[:, :K].reshape(T * K)
    gk = tg[:, :K].reshape(T * K)
    oh = (ev[:, None] == jnp.arange(E)[None, :]).astype(jnp.int32)
    csum = jnp.cumsum(oh, axis=0)
    rank = jnp.take_along_axis(csum, ev[:, None], axis=1)[:, 0] - 1
    counts = csum[-1]
    padded = ((counts + BLK - 1) // BLK) * BLK
    offs = jnp.concatenate([jnp.zeros((1,), jnp.int32),
                            jnp.cumsum(padded)[:-1].astype(jnp.int32)])
    dest = offs[ev] + rank
    src_tok = jnp.zeros((P,), jnp.int32).at[dest].set(
        jnp.arange(T * K, dtype=jnp.int32) // K)
    bstart = jnp.arange(NB, dtype=jnp.int32) * BLK
    inside = (bstart[:, None] >= offs[None, :]) & \
             (bstart[:, None] < (offs + padded)[None, :])
    be = jnp.sum(jnp.where(inside, jnp.arange(E, dtype=jnp.int32)[None, :], 0),
                 axis=1).astype(jnp.int32)
    dall = dest.reshape(T, K).T.reshape(2 * T).astype(jnp.int32)

    hg = _dispatch(h2, src_tok)
    eo = _moe(hg, be, W1.astype(jnp.bfloat16), b1.reshape(E, 1, DFF),
              W2.astype(jnp.bfloat16), b2.reshape(E, 1, D))
    R = _cgather(eo, dall)
    out = _final(x1, R, tg)
    return out.reshape(S, B, D)
